# Initial kernel scaffold; baseline (speedup 1.0000x reference)
import jax
import jax.numpy as jnp
from jax.experimental import pallas as pl


def kernel(inputs, codebook):
    raise NotImplementedError("write your pallas kernel here")


# speedup vs baseline: 1.0950x; 1.0950x over previous
import jax
import jax.numpy as jnp
from jax import lax
from jax.experimental import pallas as pl
from jax.experimental.pallas import tpu as pltpu

K = 1024
D = 64
BLK = 1024


def _vq_body(x_ref, cn_ref, cb_ref, q_ref, idx_ref, scal_ref, counts_ref, acc_ref):
    nb = pl.num_programs(0)
    pid = pl.program_id(0)

    @pl.when(pid == 0)
    def _init():
        counts_ref[...] = jnp.zeros_like(counts_ref)
        acc_ref[0, 0] = 0.0

    xb = x_ref[...]
    cb = cb_ref[...]
    xn = jnp.sum(xb * xb, axis=1, keepdims=True)
    mm = lax.dot_general(xb, cb, (((1,), (1,)), ((), ())),
                         preferred_element_type=jnp.float32)
    d = (xn + cn_ref[...]) - 2.0 * mm
    dmin = jnp.min(d, axis=1, keepdims=True)
    colid = lax.broadcasted_iota(jnp.int32, d.shape, 1)
    idx = jnp.min(jnp.where(d == dmin, colid, K), axis=1)
    idx_ref[0, 0, :] = idx
    oh = (colid == idx[:, None]).astype(jnp.float32)
    q = lax.dot_general(oh, cb, (((1,), (0,)), ((), ())),
                        preferred_element_type=jnp.float32)
    q_ref[...] = xb + (q - xb)
    counts_ref[...] += jnp.sum(oh, axis=0, keepdims=True)
    acc_ref[0, 0] += jnp.sum((q - xb) ** 2)

    @pl.when(pid == nb - 1)
    def _finalize():
        n_total = nb * BLK
        mse = acc_ref[0, 0] / (n_total * D)
        vq = 1.25 * mse
        com = 0.25 * mse
        p = counts_ref[...] / n_total
        perp = jnp.exp(-jnp.sum(p * jnp.log(p + 1e-10)))
        c = lax.broadcasted_iota(jnp.int32, (1, 128), 1)
        scal_ref[...] = jnp.where(
            c == 0, vq,
            jnp.where(c == 1, com, jnp.where(c == 2, mse,
                                             jnp.where(c == 3, perp, 0.0))))


def kernel(inputs, codebook):
    shape = inputs.shape
    x = inputs.reshape(-1, D)
    n = x.shape[0]
    nb = n // BLK
    cn_row = jnp.sum(codebook * codebook, axis=1).reshape(1, K)

    q, idx3, scal = pl.pallas_call(
        _vq_body,
        grid=(nb,),
        in_specs=[
            pl.BlockSpec((BLK, D), lambda i: (i, 0)),
            pl.BlockSpec((1, K), lambda i: (0, 0)),
            pl.BlockSpec((K, D), lambda i: (0, 0)),
        ],
        out_specs=[
            pl.BlockSpec((BLK, D), lambda i: (i, 0)),
            pl.BlockSpec((1, 1, BLK), lambda i: (i, 0, 0)),
            pl.BlockSpec((1, 128), lambda i: (0, 0)),
        ],
        out_shape=[
            jax.ShapeDtypeStruct((n, D), jnp.float32),
            jax.ShapeDtypeStruct((nb, 1, BLK), jnp.int32),
            jax.ShapeDtypeStruct((1, 128), jnp.float32),
        ],
        scratch_shapes=[
            pltpu.VMEM((1, K), jnp.float32),
            pltpu.SMEM((1, 1), jnp.float32),
        ],
    )(x, cn_row, codebook)

    quantized_st = q.reshape(shape)
    tokens = idx3.reshape(-1).reshape(shape[:-1])
    vq_loss = scal[0, 0]
    commitment_loss = scal[0, 1]
    codebook_loss = scal[0, 2]
    perplexity = scal[0, 3]
    return (quantized_st, tokens, vq_loss, commitment_loss,
            codebook_loss, perplexity)
